# Initial kernel scaffold; baseline (speedup 1.0000x reference)
import jax
import jax.numpy as jnp
from jax.experimental import pallas as pl


def kernel(x, edge_index, conv_W, conv_b, bn_gamma, bn_beta, lin_W, lin_b):
    raise NotImplementedError("write your pallas kernel here")


# speedup vs baseline: 9.5239x; 9.5239x over previous
import jax
import jax.numpy as jnp
from jax import lax
from jax.experimental import pallas as pl
from jax.experimental.pallas import tpu as pltpu
from jax.experimental.pallas import tpu_sc as plsc

N = 10000
D = 256
OUT = 64
L = 10
EPS = 1e-5

NC = 2
NS = 16
NT = NC * NS
CHUNK = 128
GS = float(1.0 / (1.0 + EPS) ** 0.5)


def _mesh():
    return plsc.VectorSubcoreMesh(core_axis_name="c", subcore_axis_name="s",
                                  num_cores=NC, num_subcores=NS)


_SC_PARAMS = pltpu.CompilerParams(needs_layout_passes=False)


def _hist_body(dst_hbm, deg_out, dst_v, hist):
    E = dst_hbm.shape[0]
    ept = E // NT
    c = lax.axis_index("c")
    s = lax.axis_index("s")
    k = c * NS + s

    pltpu.sync_copy(dst_hbm.at[pl.ds(k * ept, ept)], dst_v.at[pl.ds(0, ept)])

    zeros_f = jnp.zeros((16,), jnp.float32)
    ones_f = jnp.ones((16,), jnp.float32)
    iota = lax.iota(jnp.int32, 16)

    def init_hist(i, _):
        hist[pl.ds(i * 16, 16)] = zeros_f
        return 0
    lax.fori_loop(0, N // 16, init_hist, 0)

    def step(i, _):
        dstv = dst_v[pl.ds(i * 16, 16)]
        valid = iota < (ept - i * 16)
        dstv = jnp.where(valid, dstv, 0)
        plsc.addupdate_scatter(hist, [dstv], ones_f, mask=valid)
        return 0
    lax.fori_loop(0, (ept + 15) // 16, step, 0)

    pltpu.sync_copy(hist, deg_out.at[k])


def _make_hist(E):
    return pl.kernel(
        _hist_body,
        out_type=jax.ShapeDtypeStruct((NT, N), jnp.float32),
        mesh=_mesh(),
        compiler_params=_SC_PARAMS,
        scratch_types=[
            pltpu.VMEM((E // NT + 16,), jnp.int32),
            pltpu.VMEM((N,), jnp.float32),
        ],
    )


def _agg_body(y_hbm, src_hbm, dst_hbm, acc_hbm, idx_s, idx_d, rows, sem):
    E = src_hbm.shape[0]
    nchunks = E // CHUNK
    c = lax.axis_index("c")
    s = lax.axis_index("s")
    w = s * NC + c

    nj = lax.div(nchunks - w + (NT - 1), NT)

    def chunk(j, _):
        off = (j * NT + w) * CHUNK
        pltpu.sync_copy(src_hbm.at[pl.ds(off, CHUNK)], idx_s)
        pltpu.sync_copy(dst_hbm.at[pl.ds(off, CHUNK)], idx_d)
        pltpu.async_copy(y_hbm.at[idx_s], rows, sem).wait()
        pltpu.sync_copy(rows, acc_hbm.at[idx_d], add=True)
        return 0
    lax.fori_loop(0, nj, chunk, 0)


def _make_agg():
    return pl.kernel(
        _agg_body,
        out_type=(),
        mesh=_mesh(),
        compiler_params=_SC_PARAMS,
        scratch_types=[
            pltpu.VMEM((CHUNK,), jnp.int32),
            pltpu.VMEM((CHUNK,), jnp.int32),
            pltpu.VMEM((CHUNK, D), jnp.float32),
            pltpu.SemaphoreType.DMA,
        ],
    )


def _dinv_body(dp_ref, o_ref):
    o_ref[...] = lax.rsqrt(jnp.sum(dp_ref[...], axis=0, keepdims=True) + 1.0)


def _mm0_body(x_ref, w_ref, dinv_ref, y_ref, acc_ref):
    y = jnp.dot(x_ref[...], w_ref[...],
                preferred_element_type=jnp.float32) * dinv_ref[...]
    y_ref[...] = y
    acc_ref[...] = y


def _fused_body(acc_ref, dinv_ref, b_ref, g_ref, bt_ref, w_ref, y_ref, nacc_ref):
    dinv = dinv_ref[...]
    t = acc_ref[...] * dinv + b_ref[...]
    t = jnp.maximum(t, 0.0) * (g_ref[...] * GS) + bt_ref[...]
    y = jnp.dot(t, w_ref[...], preferred_element_type=jnp.float32) * dinv
    y_ref[...] = y
    nacc_ref[...] = y


def _final_body(acc_ref, dinv_ref, b_ref, g_ref, bt_ref, w_ref, lb_ref, o_ref):
    t = acc_ref[...] * dinv_ref[...] + b_ref[...]
    t = jnp.maximum(t, 0.0) * (g_ref[...] * GS) + bt_ref[...]
    o_ref[...] = jnp.dot(t, w_ref[...],
                         preferred_element_type=jnp.float32) + lb_ref[...]


BM = 400
GRID = (N // BM,)


def _row_spec(width):
    return pl.BlockSpec((BM, width), lambda i: (i, 0))


def _rep_spec(shape):
    return pl.BlockSpec(shape, lambda i: (0,) * len(shape))


def kernel(x, edge_index, conv_W, conv_b, bn_gamma, bn_beta, lin_W, lin_b):
    src = edge_index[0]
    dst = edge_index[1]
    E = src.shape[0]

    deg_p = _make_hist(E)(dst)

    dinv_row = pl.pallas_call(
        _dinv_body,
        out_shape=jax.ShapeDtypeStruct((1, N), jnp.float32),
    )(deg_p)
    dinv = dinv_row.reshape(N, 1)

    yy = [jax.ShapeDtypeStruct((N, D), jnp.float32)] * 2
    mm0 = pl.pallas_call(
        _mm0_body,
        grid=GRID,
        in_specs=[_row_spec(D), _rep_spec((D, D)), _row_spec(1)],
        out_specs=[_row_spec(D), _row_spec(D)],
        out_shape=yy,
    )
    fused = pl.pallas_call(
        _fused_body,
        grid=GRID,
        in_specs=[_row_spec(D), _row_spec(1), _rep_spec((1, D)),
                  _rep_spec((1, D)), _rep_spec((1, D)), _rep_spec((D, D))],
        out_specs=[_row_spec(D), _row_spec(D)],
        out_shape=yy,
    )
    final = pl.pallas_call(
        _final_body,
        grid=GRID,
        in_specs=[_row_spec(D), _row_spec(1), _rep_spec((1, D)),
                  _rep_spec((1, D)), _rep_spec((1, D)), _rep_spec((D, OUT)),
                  _rep_spec((1, OUT))],
        out_specs=_row_spec(OUT),
        out_shape=jax.ShapeDtypeStruct((N, OUT), jnp.float32),
    )
    agg_call = _make_agg()

    y, acc0 = mm0(x, conv_W[0], dinv)
    acc = acc0
    for i in range(L):
        acc_ref = jax.new_ref(acc)
        agg_call(y, src, dst, acc_ref)
        acc = acc_ref[...]
        if i < L - 1:
            y, acc = fused(acc, dinv, conv_b[i].reshape(1, D),
                           bn_gamma[i].reshape(1, D), bn_beta[i].reshape(1, D),
                           conv_W[i + 1])
        else:
            out = final(acc, dinv, conv_b[i].reshape(1, D),
                        bn_gamma[i].reshape(1, D), bn_beta[i].reshape(1, D),
                        lin_W, lin_b.reshape(1, OUT))
    return out
